# baseline (speedup 1.0000x reference)
import functools

import jax
import jax.numpy as jnp
from jax import lax
from jax.experimental import pallas as pl
from jax.experimental.pallas import tpu as pltpu
from jax.experimental.pallas import tpu_sc as plsc

NU = 50000
N = 100000
D = 64
NNZ = 1600000
BATCH = 4096

NC = 2
NS = 16
NW = NC * NS
E = NNZ // NW

NB = 8
P = N // NB
NNZ_PAD = NNZ + NW * NB * 8 + 512

SUB = 128
NSUB = 4
W = SUB * NSUB

FLUSH = 1024
STG = FLUSH + 160
SW0, SW1 = 25600, 24400

_mesh = plsc.VectorSubcoreMesh(
    core_axis_name="c", subcore_axis_name="s", num_cores=NC, num_subcores=NS)
_cparams = pltpu.CompilerParams(needs_layout_passes=False, use_tc_tiling_on_sc=False)


def _m8(x):
    return pl.multiple_of(x, 8)


def _wid():
    return lax.axis_index("c") * NS + lax.axis_index("s")


def _count_body(dst_hbm, counts_hbm, dstv, cbuf):
    wid = _wid()
    pltpu.sync_copy(dst_hbm.at[pl.ds(_m8(wid * E), E)], dstv)

    one = jnp.full((16,), 1, jnp.int32)
    zero = jnp.zeros((16,), jnp.int32)

    def body(i, carry):
        d = dstv[pl.ds(i * 16, 16)]
        return tuple(carry[k] + jnp.where(d >= (k + 1) * P, one, zero)
                     for k in range(NB - 1))

    cs = lax.fori_loop(0, E // 16, body, (zero,) * (NB - 1))
    ss = [jnp.sum(ck) for ck in cs] + [jnp.int32(0)]
    prev = jnp.int32(E)
    iota = lax.iota(jnp.int32, 16)
    out = zero
    for b in range(NB):
        out = jnp.where(iota == b, jnp.full((16,), prev - ss[b], jnp.int32),
                        out)
        prev = ss[b]
    cbuf[pl.ds(0, 16)] = out
    pltpu.sync_copy(cbuf, counts_hbm.at[wid])


_count_call = pl.kernel(
    _count_body,
    out_type=jax.ShapeDtypeStruct((NW, 16), jnp.int32),
    mesh=_mesh,
    compiler_params=_cparams,
    scratch_types=[
        pltpu.VMEM((E,), jnp.int32),
        pltpu.VMEM((16,), jnp.int32),
    ],
)


def _bucket_body(src_hbm, dst_hbm, w_hbm, counts_hbm,
                 bsrc, bdst, bw, meta,
                 cv, sstage, dstage, wstage, *rest):
    osrc = rest[0:NB]
    odst = rest[NB:2 * NB]
    oww = rest[2 * NB:3 * NB]
    metab = rest[3 * NB]
    wid = _wid()
    pltpu.sync_copy(counts_hbm, cv)

    tot = [jnp.int32(0)] * NB
    myoff = [jnp.int32(0)] * NB
    for t in range(NW):
        crow = cv[t, pl.ds(0, 16)]
        for b in range(NB):
            a = (crow[b] + 7) >> 3 << 3
            myoff[b] = myoff[b] + jnp.where(jnp.int32(t) < wid, a, 0)
            tot[b] = tot[b] + a
    bs = [jnp.int32(0)] * NB
    for b in range(1, NB):
        bs[b] = bs[b - 1] + tot[b - 1]

    iota = lax.iota(jnp.int32, 16)

    @pl.when(wid == 0)
    def _():
        for b in range(NB):
            row = jnp.where(iota == 0, jnp.full((16,), bs[b], jnp.int32),
                  jnp.where(iota == 1, jnp.full((16,), tot[b], jnp.int32), 0))
            metab[b, pl.ds(0, 16)] = row
        pltpu.sync_copy(metab, meta)

    base_chunk = wid * E
    carry = (jnp.int32(0),) * NB + tuple(bs[b] + myoff[b] for b in range(NB))

    for (w0, wlen) in ((0, SW0), (SW0, SW1)):
        pltpu.sync_copy(src_hbm.at[pl.ds(_m8(base_chunk + w0), wlen)],
                        sstage.at[pl.ds(0, wlen)])
        pltpu.sync_copy(dst_hbm.at[pl.ds(_m8(base_chunk + w0), wlen)],
                        dstage.at[pl.ds(0, wlen)])
        pltpu.sync_copy(w_hbm.at[pl.ds(_m8(base_chunk + w0), wlen)],
                        wstage.at[pl.ds(0, wlen)])

        onei = jnp.full((16,), 1, jnp.int32)
        zeroi = jnp.zeros((16,), jnp.int32)

        def do_vreg(i, fills):
            sv = sstage[pl.ds(i * 16, 16)]
            dv = dstage[pl.ds(i * 16, 16)]
            wv = wstage[pl.ds(i * 16, 16)]
            key = zeroi
            for kb in range(1, NB):
                key = key + jnp.where(dv >= kb * P, onei, zeroi)
            out = []
            for b in range(NB):
                mb = key == b
                cnt = plsc.cumsum(jnp.where(mb, onei, zeroi))
                rank = cnt - 1 + fills[b]
                plsc.store_scatter(osrc[b], [rank], sv, mask=mb)
                plsc.store_scatter(odst[b], [rank], dv - b * P, mask=mb)
                plsc.store_scatter(oww[b], [rank], wv, mask=mb)
                out.append(fills[b] + cnt[15])
            return out

        def do_flush(fb, ob, b):
            pltpu.sync_copy(osrc[b].at[pl.ds(0, FLUSH)],
                            bsrc.at[pl.ds(_m8(ob), FLUSH)])
            pltpu.sync_copy(odst[b].at[pl.ds(0, FLUSH)],
                            bdst.at[pl.ds(_m8(ob), FLUSH)])
            pltpu.sync_copy(oww[b].at[pl.ds(0, FLUSH)],
                            bw.at[pl.ds(_m8(ob), FLUSH)])
            for r in range(8):
                rs = osrc[b][pl.ds(FLUSH + r * 16, 16)]
                rd = odst[b][pl.ds(FLUSH + r * 16, 16)]
                rw = oww[b][pl.ds(FLUSH + r * 16, 16)]
                osrc[b][pl.ds(r * 16, 16)] = rs
                odst[b][pl.ds(r * 16, 16)] = rd
                oww[b][pl.ds(r * 16, 16)] = rw
            return fb - FLUSH, ob + FLUSH

        def maybe_flush(fills, offs):
            nf, no = [], []
            for b in range(NB):
                fb, ob = lax.cond(fills[b] >= FLUSH,
                                  lambda fb, ob, b=b: do_flush(fb, ob, b),
                                  lambda fb, ob: (fb, ob),
                                  fills[b], offs[b])
                nf.append(fb)
                no.append(ob)
            return nf, no

        def block_body(blk, carry):
            fills = list(carry[0:NB])
            offs = list(carry[NB:2 * NB])
            for v in range(8):
                fills = do_vreg(blk * 8 + v, fills)
            fills, offs = maybe_flush(fills, offs)
            return tuple(fills) + tuple(offs)

        carry = lax.fori_loop(0, wlen // 128, block_body, carry)

        def tail_body(i, carry):
            fills = list(carry[0:NB])
            offs = list(carry[NB:2 * NB])
            fills = do_vreg(i, fills)
            fills, offs = maybe_flush(fills, offs)
            return tuple(fills) + tuple(offs)

        if wlen % 128:
            carry = lax.fori_loop(wlen // 128 * 8, wlen // 16, tail_body,
                                  carry)

    fills = carry[0:NB]
    offs = carry[NB:2 * NB]
    zi = jnp.zeros((16,), jnp.int32)
    zf = jnp.zeros((16,), jnp.float32)
    for b in range(NB):
        osrc[b][pl.ds(fills[b], 16)] = zi
        odst[b][pl.ds(fills[b], 16)] = zi
        oww[b][pl.ds(fills[b], 16)] = zf
        n8 = (fills[b] + 7) >> 3

        def tbody(j, _, b=b, ob=offs[b]):
            pltpu.sync_copy(osrc[b].at[pl.ds(j * 8, 8)],
                            bsrc.at[pl.ds(_m8(ob + j * 8), 8)])
            pltpu.sync_copy(odst[b].at[pl.ds(j * 8, 8)],
                            bdst.at[pl.ds(_m8(ob + j * 8), 8)])
            pltpu.sync_copy(oww[b].at[pl.ds(j * 8, 8)],
                            bw.at[pl.ds(_m8(ob + j * 8), 8)])
            return 0

        lax.fori_loop(0, n8, tbody, 0)


_bucket_call = pl.kernel(
    _bucket_body,
    out_type=(
        jax.ShapeDtypeStruct((NNZ_PAD,), jnp.int32),
        jax.ShapeDtypeStruct((NNZ_PAD,), jnp.int32),
        jax.ShapeDtypeStruct((NNZ_PAD,), jnp.float32),
        jax.ShapeDtypeStruct((NB, 16), jnp.int32),
    ),
    mesh=_mesh,
    compiler_params=_cparams,
    scratch_types=[
        pltpu.VMEM((NW, 16), jnp.int32),
        pltpu.VMEM((SW0,), jnp.int32),
        pltpu.VMEM((SW0,), jnp.int32),
        pltpu.VMEM((SW0,), jnp.float32),
    ] + [pltpu.VMEM((STG,), jnp.int32)] * (2 * NB)
      + [pltpu.VMEM((STG,), jnp.float32)] * NB
      + [pltpu.VMEM((NB, 16), jnp.int32)],
)


ZROWS = 100
NZWIN = P // ZROWS


def _layer_body(x_hbm, bsrc, bdst, bw, meta,
                y_hbm, metav, sidx, didx, wvv, rows, zbuf, accum,
                isem, gsem0, gsem1, ssem0, ssem1, zsem):
    c = lax.axis_index("c")
    s = lax.axis_index("s")
    gsem = (gsem0, gsem1)
    ssem = (ssem0, ssem1)
    pltpu.sync_copy(meta, metav)

    zv = jnp.zeros((16,), jnp.float32)
    for r in range(ZROWS):
        for q in range(4):
            zbuf[r, pl.ds(q * 16, 16)] = zv

    iota = lax.iota(jnp.int32, 16)

    def pass_body(p, _):
        b = p * 2 + c
        row_base = b * P
        mrow = metav[b, pl.ds(0, 16)]
        start_b = mrow[0]
        len_b = mrow[1]
        end = start_b + len_b

        nwin = (len_b + (W - 1)) // W
        nmy = jnp.maximum(0, nwin - s + 15) // 16

        def e0_of(u):
            return start_b + (s + u * 16) * W

        def fire_idx(u, q):
            e0 = e0_of(u)
            descs = []
            for k in range(NSUB):
                descs.append(pltpu.async_copy(
                    bsrc.at[pl.ds(_m8(e0 + k * SUB), SUB)],
                    sidx.at[q, pl.ds(k * SUB, SUB)], isem))
                descs.append(pltpu.async_copy(
                    bdst.at[pl.ds(_m8(e0 + k * SUB), SUB)],
                    didx.at[q, k], isem))
                descs.append(pltpu.async_copy(
                    bw.at[pl.ds(_m8(e0 + k * SUB), SUB)],
                    wvv.at[q, pl.ds(k * SUB, SUB)], isem))
            return descs

        def wait_idx(q):
            for k in range(NSUB):
                pltpu.make_async_copy(
                    bsrc.at[pl.ds(0, SUB)],
                    sidx.at[q, pl.ds(k * SUB, SUB)], isem).wait()
                pltpu.make_async_copy(
                    bdst.at[pl.ds(0, SUB)],
                    didx.at[q, k], isem).wait()
                pltpu.make_async_copy(
                    bw.at[pl.ds(0, SUB)],
                    wvv.at[q, pl.ds(k * SUB, SUB)], isem).wait()

        def mask_idx(u, q):
            e0 = e0_of(u)

            @pl.when(e0 + W > end)
            def _():
                for kk in range(W // 16):
                    m = (e0 + kk * 16 + iota) < end
                    sl = pl.ds(kk * 16, 16)
                    sidx[q, sl] = jnp.where(m, sidx[q, sl], 0)
                    wvv[q, sl] = jnp.where(m, wvv[q, sl], 0.0)
                    dsl = pl.ds((kk % 8) * 16, 16)
                    didx[q, kk // 8, dsl] = jnp.where(
                        m, didx[q, kk // 8, dsl], 0)

        def fire_gathers(q, r2):
            for k in range(NSUB):
                pltpu.async_copy(
                    x_hbm.at[sidx.at[q, pl.ds(k * SUB, SUB)]],
                    rows.at[r2, pl.ds(k * SUB, SUB)], gsem[r2])

        def wait_gathers(q, r2):
            for k in range(NSUB):
                pltpu.make_async_copy(
                    x_hbm.at[sidx.at[q, pl.ds(k * SUB, SUB)]],
                    rows.at[r2, pl.ds(k * SUB, SUB)], gsem[r2]).wait()

        def fire_scatters(q, r2):
            for k in range(NSUB):
                pltpu.async_copy(
                    rows.at[r2, pl.ds(k * SUB, SUB)],
                    accum.at[didx.at[q, k]], ssem[r2], add=True)

        def wait_scatters(q, r2):
            for k in range(NSUB):
                pltpu.make_async_copy(
                    rows.at[r2, pl.ds(k * SUB, SUB)],
                    accum.at[didx.at[q, k]], ssem[r2]).wait()

        def multiply(q, r2):
            @plsc.parallel_loop(0, W // 16, unroll=2)
            def _(g):
                wg = wvv[q, pl.ds(g * 16, 16)]
                for l in range(16):
                    e = g * 16 + l
                    wb = jnp.full((16,), wg[l], jnp.float32)
                    for qq in range(4):
                        rows[r2, e, pl.ds(qq * 16, 16)] = (
                            rows[r2, e, pl.ds(qq * 16, 16)] * wb)

        def handle(u, q):
            r2 = q % 2

            @pl.when(u < nmy)
            def _():
                wait_gathers(q, r2)

            @pl.when((u >= 1) & (u - 1 < nmy))
            def _():
                wait_scatters((q + 3) % 4, 1 - r2)

            @pl.when(u + 1 < nmy)
            def _():
                wait_idx((q + 1) % 4)
                mask_idx(u + 1, (q + 1) % 4)
                fire_gathers((q + 1) % 4, 1 - r2)

            @pl.when(u + 2 < nmy)
            def _():
                fire_idx(u + 2, (q + 2) % 4)

            @pl.when(u < nmy)
            def _():
                multiply(q, r2)
                fire_scatters(q, r2)

        @pl.when(nmy > 0)
        def _():
            for d in fire_idx(0, 0):
                d.wait()
            mask_idx(0, 0)
            fire_gathers(0, 0)

        @pl.when(nmy > 1)
        def _():
            fire_idx(1, 1)

        for jj in range(8):
            j = s + jj * 16

            @pl.when(j < NZWIN)
            def _(j=j):
                pltpu.async_copy(zbuf, accum.at[pl.ds(j * ZROWS, ZROWS)],
                                 zsem)

        for jj in range(8):
            j = s + jj * 16

            @pl.when(j < NZWIN)
            def _(j=j):
                pltpu.make_async_copy(
                    zbuf, accum.at[pl.ds(j * ZROWS, ZROWS)], zsem).wait()

        plsc.subcore_barrier()

        def quad_body(tt, _):
            for lane in range(4):
                handle(4 * tt + lane, lane)
            return 0

        lax.fori_loop(0, (nmy + 4) // 4, quad_body, 0)
        plsc.subcore_barrier()

        for jj in range(8):
            j = s + jj * 16

            @pl.when(j < NZWIN)
            def _(j=j, row_base=row_base):
                pltpu.async_copy(
                    accum.at[pl.ds(j * ZROWS, ZROWS)],
                    y_hbm.at[pl.ds(row_base + j * ZROWS, ZROWS)], zsem)

        for jj in range(8):
            j = s + jj * 16

            @pl.when(j < NZWIN)
            def _(j=j, row_base=row_base):
                pltpu.make_async_copy(
                    accum.at[pl.ds(j * ZROWS, ZROWS)],
                    y_hbm.at[pl.ds(row_base + j * ZROWS, ZROWS)], zsem).wait()

        plsc.subcore_barrier()
        return 0

    lax.fori_loop(0, NB // 2, pass_body, 0)


_layer_call = pl.kernel(
    _layer_body,
    out_type=jax.ShapeDtypeStruct((N, D), jnp.float32),
    mesh=_mesh,
    compiler_params=_cparams,
    scratch_types=[
        pltpu.VMEM((NB, 16), jnp.int32),
        pltpu.VMEM((4, W), jnp.int32),
        pltpu.VMEM((4, NSUB, SUB), jnp.int32),
        pltpu.VMEM((4, W), jnp.float32),
        pltpu.VMEM((2, W, D), jnp.float32),
        pltpu.VMEM((ZROWS, D), jnp.float32),
        pltpu.VMEM_SHARED((P, D), jnp.float32),
        pltpu.SemaphoreType.DMA,
        pltpu.SemaphoreType.DMA,
        pltpu.SemaphoreType.DMA,
        pltpu.SemaphoreType.DMA,
        pltpu.SemaphoreType.DMA,
        pltpu.SemaphoreType.DMA,
    ],
)


BPT = BATCH // NW


def _final_body(users_hbm, items_hbm, x0, x1, x2, x3,
                gamma_hbm, uu, ii, usum, isum, gbuf):
    wid = _wid()
    pltpu.sync_copy(users_hbm.at[pl.ds(_m8(wid * BPT), BPT)], uu)
    pltpu.sync_copy(items_hbm.at[pl.ds(_m8(wid * BPT), BPT)], ii)
    for i in range(BPT // 16):
        sl = pl.ds(i * 16, 16)
        ii[sl] = ii[sl] + NU

    zv = jnp.zeros((16,), jnp.float32)

    def zbody(e, _):
        for q in range(4):
            usum[e, pl.ds(q * 16, 16)] = zv
            isum[e, pl.ds(q * 16, 16)] = zv
        return 0

    lax.fori_loop(0, BPT, zbody, 0)

    for xk in (x0, x1, x2, x3):
        pltpu.sync_copy(xk.at[uu], usum, add=True)
        pltpu.sync_copy(xk.at[ii], isum, add=True)

    iota = lax.iota(jnp.int32, 16)
    for g in range(BPT // 16):
        rowi = g * 16 + iota

        def dbody(d_, acc):
            cols = jnp.full((16,), d_, jnp.int32)
            u = plsc.load_gather(usum, [rowi, cols])
            v = plsc.load_gather(isum, [rowi, cols])
            return acc + u * v

        acc = lax.fori_loop(0, D, dbody, jnp.zeros((16,), jnp.float32))
        gbuf[pl.ds(g * 16, 16)] = acc * (1.0 / 16.0)

    pltpu.sync_copy(gbuf, gamma_hbm.at[pl.ds(_m8(wid * BPT), BPT)])


_final_call = pl.kernel(
    _final_body,
    out_type=jax.ShapeDtypeStruct((BATCH,), jnp.float32),
    mesh=_mesh,
    compiler_params=_cparams,
    scratch_types=[
        pltpu.VMEM((BPT,), jnp.int32),
        pltpu.VMEM((BPT,), jnp.int32),
        pltpu.VMEM((BPT, D), jnp.float32),
        pltpu.VMEM((BPT, D), jnp.float32),
        pltpu.VMEM((BPT,), jnp.float32),
    ],
)


def kernel(users, items, user_emb, item_emb, edge_index, edge_weight):
    src = edge_index[0]
    dst = edge_index[1]
    counts = _count_call(dst)
    bsrc, bdst, bw, meta = _bucket_call(src, dst, edge_weight, counts)
    x0 = jnp.concatenate([user_emb, item_emb], axis=0)
    x1 = _layer_call(x0, bsrc, bdst, bw, meta)
    x2 = _layer_call(x1, bsrc, bdst, bw, meta)
    x3 = _layer_call(x2, bsrc, bdst, bw, meta)
    return _final_call(users, items, x0, x1, x2, x3)


# speedup vs baseline: 1.0494x; 1.0494x over previous
import functools

import jax
import jax.numpy as jnp
from jax import lax
from jax.experimental import pallas as pl
from jax.experimental.pallas import tpu as pltpu
from jax.experimental.pallas import tpu_sc as plsc

NU = 50000
N = 100000
D = 64
NNZ = 1600000
BATCH = 4096

NC = 2
NS = 16
NW = NC * NS
E = NNZ // NW

NB = 8
P = N // NB
NNZ_PAD = NNZ + NW * NB * 8 + 512

SUB = 128
NSUB = 2
W = SUB * NSUB

FLUSH = 1024
STG = FLUSH + 160
SW0, SW1 = 25600, 24400

_mesh = plsc.VectorSubcoreMesh(
    core_axis_name="c", subcore_axis_name="s", num_cores=NC, num_subcores=NS)
_cparams = pltpu.CompilerParams(needs_layout_passes=False, use_tc_tiling_on_sc=False)


def _m8(x):
    return pl.multiple_of(x, 8)


def _wid():
    return lax.axis_index("c") * NS + lax.axis_index("s")


def _count_body(dst_hbm, counts_hbm, dstv, cbuf):
    wid = _wid()
    pltpu.sync_copy(dst_hbm.at[pl.ds(_m8(wid * E), E)], dstv)

    one = jnp.full((16,), 1, jnp.int32)
    zero = jnp.zeros((16,), jnp.int32)

    def body(i, carry):
        d = dstv[pl.ds(i * 16, 16)]
        return tuple(carry[k] + jnp.where(d >= (k + 1) * P, one, zero)
                     for k in range(NB - 1))

    cs = lax.fori_loop(0, E // 16, body, (zero,) * (NB - 1))
    ss = [jnp.sum(ck) for ck in cs] + [jnp.int32(0)]
    prev = jnp.int32(E)
    iota = lax.iota(jnp.int32, 16)
    out = zero
    for b in range(NB):
        out = jnp.where(iota == b, jnp.full((16,), prev - ss[b], jnp.int32),
                        out)
        prev = ss[b]
    cbuf[pl.ds(0, 16)] = out
    pltpu.sync_copy(cbuf, counts_hbm.at[wid])


_count_call = pl.kernel(
    _count_body,
    out_type=jax.ShapeDtypeStruct((NW, 16), jnp.int32),
    mesh=_mesh,
    compiler_params=_cparams,
    scratch_types=[
        pltpu.VMEM((E,), jnp.int32),
        pltpu.VMEM((16,), jnp.int32),
    ],
)


def _bucket_body(src_hbm, dst_hbm, w_hbm, counts_hbm,
                 bsrc, bdst, bw, meta,
                 cv, sstage, dstage, wstage, *rest):
    osrc = rest[0:NB]
    odst = rest[NB:2 * NB]
    oww = rest[2 * NB:3 * NB]
    metab = rest[3 * NB]
    wid = _wid()
    pltpu.sync_copy(counts_hbm, cv)

    tot = [jnp.int32(0)] * NB
    myoff = [jnp.int32(0)] * NB
    for t in range(NW):
        crow = cv[t, pl.ds(0, 16)]
        for b in range(NB):
            a = (crow[b] + 7) >> 3 << 3
            myoff[b] = myoff[b] + jnp.where(jnp.int32(t) < wid, a, 0)
            tot[b] = tot[b] + a
    bs = [jnp.int32(0)] * NB
    for b in range(1, NB):
        bs[b] = bs[b - 1] + tot[b - 1]

    iota = lax.iota(jnp.int32, 16)

    @pl.when(wid == 0)
    def _():
        for b in range(NB):
            row = jnp.where(iota == 0, jnp.full((16,), bs[b], jnp.int32),
                  jnp.where(iota == 1, jnp.full((16,), tot[b], jnp.int32), 0))
            metab[b, pl.ds(0, 16)] = row
        pltpu.sync_copy(metab, meta)

    base_chunk = wid * E
    carry = (jnp.int32(0),) * NB + tuple(bs[b] + myoff[b] for b in range(NB))

    for (w0, wlen) in ((0, SW0), (SW0, SW1)):
        pltpu.sync_copy(src_hbm.at[pl.ds(_m8(base_chunk + w0), wlen)],
                        sstage.at[pl.ds(0, wlen)])
        pltpu.sync_copy(dst_hbm.at[pl.ds(_m8(base_chunk + w0), wlen)],
                        dstage.at[pl.ds(0, wlen)])
        pltpu.sync_copy(w_hbm.at[pl.ds(_m8(base_chunk + w0), wlen)],
                        wstage.at[pl.ds(0, wlen)])

        onei = jnp.full((16,), 1, jnp.int32)
        zeroi = jnp.zeros((16,), jnp.int32)

        def do_vreg(i, fills):
            sv = sstage[pl.ds(i * 16, 16)]
            dv = dstage[pl.ds(i * 16, 16)]
            wv = wstage[pl.ds(i * 16, 16)]
            key = zeroi
            for kb in range(1, NB):
                key = key + jnp.where(dv >= kb * P, onei, zeroi)
            out = []
            for b in range(NB):
                mb = key == b
                cnt = plsc.cumsum(jnp.where(mb, onei, zeroi))
                rank = cnt - 1 + fills[b]
                plsc.store_scatter(osrc[b], [rank], sv, mask=mb)
                plsc.store_scatter(odst[b], [rank], dv - b * P, mask=mb)
                plsc.store_scatter(oww[b], [rank], wv, mask=mb)
                out.append(fills[b] + cnt[15])
            return out

        def do_flush(fb, ob, b):
            pltpu.sync_copy(osrc[b].at[pl.ds(0, FLUSH)],
                            bsrc.at[pl.ds(_m8(ob), FLUSH)])
            pltpu.sync_copy(odst[b].at[pl.ds(0, FLUSH)],
                            bdst.at[pl.ds(_m8(ob), FLUSH)])
            pltpu.sync_copy(oww[b].at[pl.ds(0, FLUSH)],
                            bw.at[pl.ds(_m8(ob), FLUSH)])
            for r in range(8):
                rs = osrc[b][pl.ds(FLUSH + r * 16, 16)]
                rd = odst[b][pl.ds(FLUSH + r * 16, 16)]
                rw = oww[b][pl.ds(FLUSH + r * 16, 16)]
                osrc[b][pl.ds(r * 16, 16)] = rs
                odst[b][pl.ds(r * 16, 16)] = rd
                oww[b][pl.ds(r * 16, 16)] = rw
            return fb - FLUSH, ob + FLUSH

        def maybe_flush(fills, offs):
            nf, no = [], []
            for b in range(NB):
                fb, ob = lax.cond(fills[b] >= FLUSH,
                                  lambda fb, ob, b=b: do_flush(fb, ob, b),
                                  lambda fb, ob: (fb, ob),
                                  fills[b], offs[b])
                nf.append(fb)
                no.append(ob)
            return nf, no

        def block_body(blk, carry):
            fills = list(carry[0:NB])
            offs = list(carry[NB:2 * NB])
            for v in range(8):
                fills = do_vreg(blk * 8 + v, fills)
            fills, offs = maybe_flush(fills, offs)
            return tuple(fills) + tuple(offs)

        carry = lax.fori_loop(0, wlen // 128, block_body, carry)

        def tail_body(i, carry):
            fills = list(carry[0:NB])
            offs = list(carry[NB:2 * NB])
            fills = do_vreg(i, fills)
            fills, offs = maybe_flush(fills, offs)
            return tuple(fills) + tuple(offs)

        if wlen % 128:
            carry = lax.fori_loop(wlen // 128 * 8, wlen // 16, tail_body,
                                  carry)

    fills = carry[0:NB]
    offs = carry[NB:2 * NB]
    zi = jnp.zeros((16,), jnp.int32)
    zf = jnp.zeros((16,), jnp.float32)
    for b in range(NB):
        osrc[b][pl.ds(fills[b], 16)] = zi
        odst[b][pl.ds(fills[b], 16)] = zi
        oww[b][pl.ds(fills[b], 16)] = zf
        n8 = (fills[b] + 7) >> 3

        def tbody(j, _, b=b, ob=offs[b]):
            pltpu.sync_copy(osrc[b].at[pl.ds(j * 8, 8)],
                            bsrc.at[pl.ds(_m8(ob + j * 8), 8)])
            pltpu.sync_copy(odst[b].at[pl.ds(j * 8, 8)],
                            bdst.at[pl.ds(_m8(ob + j * 8), 8)])
            pltpu.sync_copy(oww[b].at[pl.ds(j * 8, 8)],
                            bw.at[pl.ds(_m8(ob + j * 8), 8)])
            return 0

        lax.fori_loop(0, n8, tbody, 0)


_bucket_call = pl.kernel(
    _bucket_body,
    out_type=(
        jax.ShapeDtypeStruct((NNZ_PAD,), jnp.int32),
        jax.ShapeDtypeStruct((NNZ_PAD,), jnp.int32),
        jax.ShapeDtypeStruct((NNZ_PAD,), jnp.float32),
        jax.ShapeDtypeStruct((NB, 16), jnp.int32),
    ),
    mesh=_mesh,
    compiler_params=_cparams,
    scratch_types=[
        pltpu.VMEM((NW, 16), jnp.int32),
        pltpu.VMEM((SW0,), jnp.int32),
        pltpu.VMEM((SW0,), jnp.int32),
        pltpu.VMEM((SW0,), jnp.float32),
    ] + [pltpu.VMEM((STG,), jnp.int32)] * (2 * NB)
      + [pltpu.VMEM((STG,), jnp.float32)] * NB
      + [pltpu.VMEM((NB, 16), jnp.int32)],
)


ZROWS = 100
NZWIN = P // ZROWS


def _layer_body(x_hbm, bsrc, bdst, bw, meta,
                y_hbm, metav, sidx, didx, wvv, rows, zbuf, accum,
                isem, gsem0, gsem1, gsem2, gsem3, ssem0, ssem1, ssem2, ssem3,
                zsem):
    c = lax.axis_index("c")
    s = lax.axis_index("s")
    gsem = (gsem0, gsem1, gsem2, gsem3)
    ssem = (ssem0, ssem1, ssem2, ssem3)
    pltpu.sync_copy(meta, metav)

    zv = jnp.zeros((16,), jnp.float32)
    for r in range(ZROWS):
        for q in range(4):
            zbuf[r, pl.ds(q * 16, 16)] = zv

    iota = lax.iota(jnp.int32, 16)

    def pass_body(p, _):
        b = p * 2 + c
        row_base = b * P
        mrow = metav[b, pl.ds(0, 16)]
        start_b = mrow[0]
        len_b = mrow[1]
        end = start_b + len_b

        nwin = (len_b + (W - 1)) // W
        nmy = jnp.maximum(0, nwin - s + 15) // 16

        def e0_of(u):
            return start_b + (s + u * 16) * W

        def fire_idx(u, q):
            e0 = e0_of(u)
            descs = []
            for k in range(NSUB):
                descs.append(pltpu.async_copy(
                    bsrc.at[pl.ds(_m8(e0 + k * SUB), SUB)],
                    sidx.at[q, pl.ds(k * SUB, SUB)], isem))
                descs.append(pltpu.async_copy(
                    bdst.at[pl.ds(_m8(e0 + k * SUB), SUB)],
                    didx.at[q, k], isem))
                descs.append(pltpu.async_copy(
                    bw.at[pl.ds(_m8(e0 + k * SUB), SUB)],
                    wvv.at[q, pl.ds(k * SUB, SUB)], isem))
            return descs

        def wait_idx(q):
            for k in range(NSUB):
                pltpu.make_async_copy(
                    bsrc.at[pl.ds(0, SUB)],
                    sidx.at[q, pl.ds(k * SUB, SUB)], isem).wait()
                pltpu.make_async_copy(
                    bdst.at[pl.ds(0, SUB)],
                    didx.at[q, k], isem).wait()
                pltpu.make_async_copy(
                    bw.at[pl.ds(0, SUB)],
                    wvv.at[q, pl.ds(k * SUB, SUB)], isem).wait()

        def mask_idx(u, q):
            e0 = e0_of(u)

            @pl.when(e0 + W > end)
            def _():
                for kk in range(W // 16):
                    m = (e0 + kk * 16 + iota) < end
                    sl = pl.ds(kk * 16, 16)
                    sidx[q, sl] = jnp.where(m, sidx[q, sl], 0)
                    wvv[q, sl] = jnp.where(m, wvv[q, sl], 0.0)
                    dsl = pl.ds((kk % 8) * 16, 16)
                    didx[q, kk // 8, dsl] = jnp.where(
                        m, didx[q, kk // 8, dsl], 0)

        def fire_gathers(q):
            for k in range(NSUB):
                pltpu.async_copy(
                    x_hbm.at[sidx.at[q, pl.ds(k * SUB, SUB)]],
                    rows.at[q, pl.ds(k * SUB, SUB)], gsem[q])

        def wait_gathers(q):
            for k in range(NSUB):
                pltpu.make_async_copy(
                    x_hbm.at[sidx.at[q, pl.ds(k * SUB, SUB)]],
                    rows.at[q, pl.ds(k * SUB, SUB)], gsem[q]).wait()

        def fire_scatters(q):
            for k in range(NSUB):
                pltpu.async_copy(
                    rows.at[q, pl.ds(k * SUB, SUB)],
                    accum.at[didx.at[q, k]], ssem[q], add=True)

        def wait_scatters(q):
            for k in range(NSUB):
                pltpu.make_async_copy(
                    rows.at[q, pl.ds(k * SUB, SUB)],
                    accum.at[didx.at[q, k]], ssem[q]).wait()

        def multiply(q):
            @plsc.parallel_loop(0, W // 16, unroll=2)
            def _(g):
                wg = wvv[q, pl.ds(g * 16, 16)]
                for l in range(16):
                    e = g * 16 + l
                    wb = jnp.full((16,), wg[l], jnp.float32)
                    for qq in range(4):
                        rows[q, e, pl.ds(qq * 16, 16)] = (
                            rows[q, e, pl.ds(qq * 16, 16)] * wb)

        def handle(u, q):
            @pl.when(u < nmy)
            def _():
                wait_gathers(q)

            @pl.when((u >= 2) & (u - 2 < nmy))
            def _():
                wait_scatters((q + 2) % 4)

            @pl.when(u + 1 < nmy)
            def _():
                wait_idx((q + 1) % 4)
                mask_idx(u + 1, (q + 1) % 4)
                fire_gathers((q + 1) % 4)

            @pl.when(u + 2 < nmy)
            def _():
                fire_idx(u + 2, (q + 2) % 4)

            @pl.when(u < nmy)
            def _():
                multiply(q)
                fire_scatters(q)

        @pl.when(nmy > 0)
        def _():
            for d in fire_idx(0, 0):
                d.wait()
            mask_idx(0, 0)
            fire_gathers(0)

        @pl.when(nmy > 1)
        def _():
            fire_idx(1, 1)

        for jj in range(8):
            j = s + jj * 16

            @pl.when(j < NZWIN)
            def _(j=j):
                pltpu.async_copy(zbuf, accum.at[pl.ds(j * ZROWS, ZROWS)],
                                 zsem)

        for jj in range(8):
            j = s + jj * 16

            @pl.when(j < NZWIN)
            def _(j=j):
                pltpu.make_async_copy(
                    zbuf, accum.at[pl.ds(j * ZROWS, ZROWS)], zsem).wait()

        plsc.subcore_barrier()

        def quad_body(tt, _):
            for lane in range(4):
                handle(4 * tt + lane, lane)
            return 0

        lax.fori_loop(0, (nmy + 5) // 4, quad_body, 0)
        plsc.subcore_barrier()

        for jj in range(8):
            j = s + jj * 16

            @pl.when(j < NZWIN)
            def _(j=j, row_base=row_base):
                pltpu.async_copy(
                    accum.at[pl.ds(j * ZROWS, ZROWS)],
                    y_hbm.at[pl.ds(row_base + j * ZROWS, ZROWS)], zsem)

        for jj in range(8):
            j = s + jj * 16

            @pl.when(j < NZWIN)
            def _(j=j, row_base=row_base):
                pltpu.make_async_copy(
                    accum.at[pl.ds(j * ZROWS, ZROWS)],
                    y_hbm.at[pl.ds(row_base + j * ZROWS, ZROWS)], zsem).wait()

        plsc.subcore_barrier()
        return 0

    lax.fori_loop(0, NB // 2, pass_body, 0)


_layer_call = pl.kernel(
    _layer_body,
    out_type=jax.ShapeDtypeStruct((N, D), jnp.float32),
    mesh=_mesh,
    compiler_params=_cparams,
    scratch_types=[
        pltpu.VMEM((NB, 16), jnp.int32),
        pltpu.VMEM((4, W), jnp.int32),
        pltpu.VMEM((4, NSUB, SUB), jnp.int32),
        pltpu.VMEM((4, W), jnp.float32),
        pltpu.VMEM((4, W, D), jnp.float32),
        pltpu.VMEM((ZROWS, D), jnp.float32),
        pltpu.VMEM_SHARED((P, D), jnp.float32),
    ] + [pltpu.SemaphoreType.DMA] * 10,
)


BPT = BATCH // NW


def _final_body(users_hbm, items_hbm, x0, x1, x2, x3,
                gamma_hbm, uu, ii, usum, isum, gbuf):
    wid = _wid()
    pltpu.sync_copy(users_hbm.at[pl.ds(_m8(wid * BPT), BPT)], uu)
    pltpu.sync_copy(items_hbm.at[pl.ds(_m8(wid * BPT), BPT)], ii)
    for i in range(BPT // 16):
        sl = pl.ds(i * 16, 16)
        ii[sl] = ii[sl] + NU

    zv = jnp.zeros((16,), jnp.float32)

    def zbody(e, _):
        for q in range(4):
            usum[e, pl.ds(q * 16, 16)] = zv
            isum[e, pl.ds(q * 16, 16)] = zv
        return 0

    lax.fori_loop(0, BPT, zbody, 0)

    for xk in (x0, x1, x2, x3):
        pltpu.sync_copy(xk.at[uu], usum, add=True)
        pltpu.sync_copy(xk.at[ii], isum, add=True)

    iota = lax.iota(jnp.int32, 16)
    for g in range(BPT // 16):
        rowi = g * 16 + iota

        def dbody(d_, acc):
            cols = jnp.full((16,), d_, jnp.int32)
            u = plsc.load_gather(usum, [rowi, cols])
            v = plsc.load_gather(isum, [rowi, cols])
            return acc + u * v

        acc = lax.fori_loop(0, D, dbody, jnp.zeros((16,), jnp.float32))
        gbuf[pl.ds(g * 16, 16)] = acc * (1.0 / 16.0)

    pltpu.sync_copy(gbuf, gamma_hbm.at[pl.ds(_m8(wid * BPT), BPT)])


_final_call = pl.kernel(
    _final_body,
    out_type=jax.ShapeDtypeStruct((BATCH,), jnp.float32),
    mesh=_mesh,
    compiler_params=_cparams,
    scratch_types=[
        pltpu.VMEM((BPT,), jnp.int32),
        pltpu.VMEM((BPT,), jnp.int32),
        pltpu.VMEM((BPT, D), jnp.float32),
        pltpu.VMEM((BPT, D), jnp.float32),
        pltpu.VMEM((BPT,), jnp.float32),
    ],
)


def kernel(users, items, user_emb, item_emb, edge_index, edge_weight):
    src = edge_index[0]
    dst = edge_index[1]
    counts = _count_call(dst)
    bsrc, bdst, bw, meta = _bucket_call(src, dst, edge_weight, counts)
    x0 = jnp.concatenate([user_emb, item_emb], axis=0)
    x1 = _layer_call(x0, bsrc, bdst, bw, meta)
    x2 = _layer_call(x1, bsrc, bdst, bw, meta)
    x3 = _layer_call(x2, bsrc, bdst, bw, meta)
    return _final_call(users, items, x0, x1, x2, x3)
